# baseline (speedup 1.0000x reference)
import jax
import jax.numpy as jnp
import numpy as np
from jax.experimental import pallas as pl
from jax.experimental.pallas import tpu as pltpu

R = 128
V = 100000
C_BLK = 2560
NB = (V + C_BLK - 1) // C_BLK

_TINY = np.float32(np.finfo(np.float32).tiny)

_KS0 = np.uint32(0)
_KS1 = np.uint32(42)
_KS2 = np.uint32(np.uint32(0x1BD11BDA) ^ _KS0 ^ _KS1)

_ROT_A = (13, 15, 26, 6)
_ROT_B = (17, 29, 16, 24)

_NEG_INF = np.float32(-np.inf)
_IMAX = np.int32(2**31 - 1)


def _rotl(x, d):
    return (x << np.uint32(d)) | (x >> np.uint32(32 - d))


def _threefry_bits(n):
    x0 = jnp.zeros_like(n) + _KS0
    x1 = n + _KS1

    def rounds(x0, x1, rots):
        for r in rots:
            x0 = x0 + x1
            x1 = _rotl(x1, r)
            x1 = x0 ^ x1
        return x0, x1

    x0, x1 = rounds(x0, x1, _ROT_A)
    x0, x1 = x0 + _KS1, x1 + (_KS2 + np.uint32(1))
    x0, x1 = rounds(x0, x1, _ROT_B)
    x0, x1 = x0 + _KS2, x1 + (_KS0 + np.uint32(2))
    x0, x1 = rounds(x0, x1, _ROT_A)
    x0, x1 = x0 + _KS0, x1 + (_KS1 + np.uint32(3))
    x0, x1 = rounds(x0, x1, _ROT_B)
    x0, x1 = x0 + _KS1, x1 + (_KS2 + np.uint32(4))
    x0, x1 = rounds(x0, x1, _ROT_A)
    x0, x1 = x0 + _KS2, x1 + (_KS0 + np.uint32(5))
    return x0 ^ x1


def _gumbel_block(col0_u32):
    rows = jax.lax.broadcasted_iota(jnp.uint32, (R, C_BLK), 0)
    cols = col0_u32 + jax.lax.broadcasted_iota(jnp.uint32, (R, C_BLK), 1)
    n = rows * np.uint32(V) + cols
    bits = _threefry_bits(n)
    float_bits = (bits >> np.uint32(9)) | np.uint32(0x3F800000)
    u = jax.lax.bitcast_convert_type(float_bits, jnp.float32) - np.float32(1.0)
    u = jnp.maximum(_TINY, u)
    return -jnp.log(-jnp.log(u))


def _sample_kernel(logits_ref, temp_ref, out_ref, best_val, best_idx):
    b = pl.program_id(0)
    t = temp_ref[0, 0]
    col0 = (b * C_BLK).astype(jnp.int32)
    g = _gumbel_block(col0.astype(jnp.uint32))
    val = g + logits_ref[...] / t
    cols = col0 + jax.lax.broadcasted_iota(jnp.int32, (R, C_BLK), 1)
    val = jnp.where(cols < V, val, _NEG_INF)

    local_max = jnp.max(val, axis=1, keepdims=True)
    at_max = val == local_max
    idx_or_big = jnp.where(at_max, cols, _IMAX)
    local_arg = jnp.min(idx_or_big, axis=1, keepdims=True)

    @pl.when(b == 0)
    def _init():
        best_val[...] = local_max
        best_idx[...] = local_arg

    @pl.when(b > 0)
    def _merge():
        better = local_max > best_val[...]
        best_idx[...] = jnp.where(better, local_arg, best_idx[...])
        best_val[...] = jnp.where(better, local_max, best_val[...])

    @pl.when(b == NB - 1)
    def _emit():
        out_ref[...] = best_idx[...]


@jax.jit
def _sample(logits, temp2d):
    out = pl.pallas_call(
        _sample_kernel,
        grid=(NB,),
        in_specs=[
            pl.BlockSpec((R, C_BLK), lambda b: (0, b)),
            pl.BlockSpec((1, 1), lambda b: (0, 0)),
        ],
        out_specs=pl.BlockSpec((R, 1), lambda b: (0, 0)),
        out_shape=jax.ShapeDtypeStruct((R, 1), jnp.int32),
        scratch_shapes=[
            pltpu.VMEM((R, 1), jnp.float32),
            pltpu.VMEM((R, 1), jnp.int32),
        ],
        compiler_params=pltpu.CompilerParams(
            dimension_semantics=("arbitrary",),
        ),
    )(logits, temp2d)
    return out.reshape(R)


def kernel(logits, temperature):
    return _sample(logits, temperature.reshape(1, 1))


# speedup vs baseline: 1.0341x; 1.0341x over previous
import jax
import jax.numpy as jnp
import numpy as np
from jax.experimental import pallas as pl
from jax.experimental.pallas import tpu as pltpu

R = 128
V = 100000
C_BLK = 2048
NB = (V + C_BLK - 1) // C_BLK

_TINY = np.float32(np.finfo(np.float32).tiny)

_KS0 = np.uint32(0)
_KS1 = np.uint32(42)
_KS2 = np.uint32(np.uint32(0x1BD11BDA) ^ _KS0 ^ _KS1)

_ROT_A = (13, 15, 26, 6)
_ROT_B = (17, 29, 16, 24)

_NEG_INF = np.float32(-np.inf)
_IMAX = np.int32(2**31 - 1)


def _rotl(x, d):
    return (x << np.uint32(d)) | (x >> np.uint32(32 - d))


def _threefry_bits(n):
    x0 = jnp.zeros_like(n) + _KS0
    x1 = n + _KS1

    def rounds(x0, x1, rots):
        for r in rots:
            x0 = x0 + x1
            x1 = _rotl(x1, r)
            x1 = x0 ^ x1
        return x0, x1

    x0, x1 = rounds(x0, x1, _ROT_A)
    x0, x1 = x0 + _KS1, x1 + (_KS2 + np.uint32(1))
    x0, x1 = rounds(x0, x1, _ROT_B)
    x0, x1 = x0 + _KS2, x1 + (_KS0 + np.uint32(2))
    x0, x1 = rounds(x0, x1, _ROT_A)
    x0, x1 = x0 + _KS0, x1 + (_KS1 + np.uint32(3))
    x0, x1 = rounds(x0, x1, _ROT_B)
    x0, x1 = x0 + _KS1, x1 + (_KS2 + np.uint32(4))
    x0, x1 = rounds(x0, x1, _ROT_A)
    x0, x1 = x0 + _KS2, x1 + (_KS0 + np.uint32(5))
    return x0 ^ x1


def _gumbel_block(col0_u32):
    rows = jax.lax.broadcasted_iota(jnp.uint32, (R, C_BLK), 0)
    cols = col0_u32 + jax.lax.broadcasted_iota(jnp.uint32, (R, C_BLK), 1)
    n = rows * np.uint32(V) + cols
    bits = _threefry_bits(n)
    float_bits = (bits >> np.uint32(9)) | np.uint32(0x3F800000)
    u = jax.lax.bitcast_convert_type(float_bits, jnp.float32) - np.float32(1.0)
    u = jnp.maximum(_TINY, u)
    return -jnp.log(-jnp.log(u))


def _sample_kernel(logits_ref, temp_ref, out_ref, best_val, best_idx):
    b = pl.program_id(0)
    t = temp_ref[0, 0]
    col0 = (b * C_BLK).astype(jnp.int32)
    g = _gumbel_block(col0.astype(jnp.uint32))
    val = g + logits_ref[...] / t
    cols = col0 + jax.lax.broadcasted_iota(jnp.int32, (R, C_BLK), 1)
    val = jnp.where(cols < V, val, _NEG_INF)

    local_max = jnp.max(val, axis=1, keepdims=True)
    at_max = val == local_max
    idx_or_big = jnp.where(at_max, cols, _IMAX)
    local_arg = jnp.min(idx_or_big, axis=1, keepdims=True)

    @pl.when(b == 0)
    def _init():
        best_val[...] = local_max
        best_idx[...] = local_arg

    @pl.when(b > 0)
    def _merge():
        better = local_max > best_val[...]
        best_idx[...] = jnp.where(better, local_arg, best_idx[...])
        best_val[...] = jnp.where(better, local_max, best_val[...])

    @pl.when(b == NB - 1)
    def _emit():
        out_ref[...] = best_idx[...]


@jax.jit
def _sample(logits, temp2d):
    out = pl.pallas_call(
        _sample_kernel,
        grid=(NB,),
        in_specs=[
            pl.BlockSpec((R, C_BLK), lambda b: (0, b)),
            pl.BlockSpec((1, 1), lambda b: (0, 0)),
        ],
        out_specs=pl.BlockSpec((R, 1), lambda b: (0, 0)),
        out_shape=jax.ShapeDtypeStruct((R, 1), jnp.int32),
        scratch_shapes=[
            pltpu.VMEM((R, 1), jnp.float32),
            pltpu.VMEM((R, 1), jnp.int32),
        ],
        compiler_params=pltpu.CompilerParams(
            dimension_semantics=("arbitrary",),
        ),
    )(logits, temp2d)
    return out.reshape(R)


def kernel(logits, temperature):
    return _sample(logits, temperature.reshape(1, 1))
